# baseline (speedup 1.0000x reference)
import functools

import jax
import jax.numpy as jnp
from jax import lax
from jax.experimental import pallas as pl
from jax.experimental.pallas import tpu as pltpu
from jax.experimental.pallas import tpu_sc as plsc

J = 19
P = 200
PPAD = 208
NVREG = PPAD // 16
KNN_VOLS = 3
K = 8
Q = 16384
NC, NS, L = 2, 16, 16
NW = NC * NS
QW = Q // NW
PAD_VAL = 1.0e15
BIGF = 3.0e38


def _min16(a, b):
  ak, av = a
  bk, bv = b
  take_a = ak <= bk
  return jnp.where(take_a, ak, bk), jnp.where(take_a, av, bv)


def _top16(items, desc):
  if len(items) == 1:
    return plsc.sort_key_val(items[0][0], items[0][1], descending=desc)
  mid = (len(items) + 1) // 2
  a = _top16(items[:mid], False)
  b = _top16(items[mid:], True)
  mk, mv = _min16(a, b)
  return plsc.sort_key_val(mk, mv, descending=desc)


_mesh = plsc.VectorSubcoreMesh(
    core_axis_name="c", subcore_axis_name="s", num_cores=NC, num_subcores=NS)


@functools.partial(
    pl.kernel,
    out_type=(jax.ShapeDtypeStruct((Q * K,), jnp.float32),
              jax.ShapeDtypeStruct((Q * K,), jnp.int32)),
    mesh=_mesh,
    compiler_params=pltpu.CompilerParams(needs_layout_passes=False),
    scratch_types=[
        pltpu.VMEM((QW, 3), jnp.float32),
        pltpu.VMEM((3, J, PPAD), jnp.float32),
        pltpu.VMEM((32, 3), jnp.float32),
        pltpu.VMEM((QW * K + 8,), jnp.float32),
        pltpu.VMEM((QW * K + 8,), jnp.int32),
    ])
def _knn_sc(q_hbm, p_hbm, s_hbm, kd_hbm, ki_hbm,
            q_v, p_v, s_v, kd_v, ki_v):
  wid = lax.axis_index("s") * NC + lax.axis_index("c")
  qbase = wid * QW
  pltpu.sync_copy(q_hbm.at[pl.ds(qbase, QW)], q_v)
  pltpu.sync_copy(p_hbm, p_v)
  pltpu.sync_copy(s_hbm, s_v)

  iota = lax.iota(jnp.int32, L)
  cvec = [jnp.full((L,), c, jnp.int32) for c in range(3)]

  def scale_body(j, carry):
    jvec = jnp.full((L,), j, jnp.int32)
    for c in range(3):
      sc = plsc.load_gather(s_v, [jvec, cvec[c]])
      for k in range(NVREG):
        sl = pl.ds(k * 16, 16)
        p_v[c, j, sl] = p_v[c, j, sl] / sc
    return carry

  lax.fori_loop(0, J, scale_body, 0)

  zero = jnp.zeros((L,), jnp.float32)

  def ctr_body(j, carry):
    carry = list(carry)
    for c in range(3):
      acc = p_v[c, j, pl.ds(0, 16)]
      for k in range(1, NVREG - 1):
        acc = acc + p_v[c, j, pl.ds(k * 16, 16)]
      last = jnp.where(iota < 8, p_v[c, j, pl.ds((NVREG - 1) * 16, 16)], 0.0)
      acc = acc + last
      m = jnp.sum(acc) * jnp.float32(1.0 / P)
      for h in range(2):
        carry[c * 2 + h] = jnp.where(iota + h * 16 == j, m, carry[c * 2 + h])
    return tuple(carry)

  ctr = lax.fori_loop(0, J, ctr_body, (zero,) * 6)

  def q_body(i, carry):
    ivec = jnp.full((L,), i, jnp.int32)
    q = [plsc.load_gather(q_v, [ivec, cvec[c]]) for c in range(3)]

    halves = []
    for h in range(2):
      jlane = iota + h * 16
      d = None
      for c in range(3):
        sc = plsc.load_gather(s_v, [jlane, cvec[c]])
        t = q[c] / sc - ctr[c * 2 + h]
        d = t * t if d is None else d + t * t
      if h == 1:
        d = jnp.where(jlane < J, d, BIGF)
      halves.append((d, jlane))
    _, v3 = _top16(halves, False)

    runs = []
    for r in range(KNN_VOLS):
      jr_s = jnp.sum(jnp.where(iota == r, v3, 0))
      jrv = jnp.full((L,), jr_s, jnp.int32)
      qs = [q[c] / plsc.load_gather(s_v, [jrv, cvec[c]]) for c in range(3)]
      gbase = jrv * P
      for k in range(NVREG):
        sp = iota + k * 16
        d = None
        for c in range(3):
          g = p_v[c, jr_s, pl.ds(k * 16, 16)]
          t = qs[c] - g
          d = t * t if d is None else d + t * t
        runs.append((d, gbase + sp))
    bk, bv = _top16(runs, False)

    msk = iota < K
    plsc.store_compressed(kd_v.at[pl.ds(i * K, 16)], bk, mask=msk)
    plsc.store_compressed(ki_v.at[pl.ds(i * K, 16)], bv, mask=msk)
    return carry

  def q_pair(i2, carry):
    q_body(i2 * 2, carry)
    q_body(i2 * 2 + 1, carry)
    return carry

  lax.fori_loop(0, QW // 2, q_pair, 0)

  pltpu.sync_copy(kd_v.at[pl.ds(0, QW * K)], kd_hbm.at[pl.ds(qbase * K, QW * K)])
  pltpu.sync_copy(ki_v.at[pl.ds(0, QW * K)], ki_hbm.at[pl.ds(qbase * K, QW * K)])


def kernel(q_w, p_w, vol_scale):
  p_pad = jnp.full((3, J, PPAD), PAD_VAL, jnp.float32)
  p_pad = p_pad.at[:, :, :P].set(jnp.transpose(p_w, (2, 0, 1)))
  s_pad = jnp.ones((32, 3), jnp.float32).at[:J, :].set(vol_scale)
  kd, ki = _knn_sc(q_w, p_pad, s_pad)
  return kd.reshape(Q, K), ki.reshape(Q, K)


# speedup vs baseline: 1.5375x; 1.5375x over previous
import functools

import jax
import jax.numpy as jnp
from jax import lax
from jax.experimental import pallas as pl
from jax.experimental.pallas import tpu as pltpu
from jax.experimental.pallas import tpu_sc as plsc

J = 19
P = 200
PPAD = 208
NVREG = PPAD // 16
KNN_VOLS = 3
K = 8
Q = 16384
NC, NS, L = 2, 16, 16
NW = NC * NS
QW = Q // NW
PAD_VAL = 1.0e15
BIGF = 3.0e38


def _merge16(a, b):
  ak, av = a
  bk, bv = b
  rk = lax.rev(bk, (0,))
  rv = lax.rev(bv, (0,))
  take_a = ak <= rk
  mk = jnp.where(take_a, ak, rk)
  mv = jnp.where(take_a, av, rv)
  sk, sv = plsc.sort_key_val(mk, mv)
  return sk, sv


def _top16(items):
  runs = [plsc.sort_key_val(k, v) for k, v in items]
  while len(runs) > 1:
    nxt = [_merge16(runs[t], runs[t + 1]) for t in range(0, len(runs) - 1, 2)]
    if len(runs) % 2:
      nxt.append(runs[-1])
    runs = nxt
  return runs[0]


_mesh = plsc.VectorSubcoreMesh(
    core_axis_name="c", subcore_axis_name="s", num_cores=NC, num_subcores=NS)


@functools.partial(
    pl.kernel,
    out_type=(jax.ShapeDtypeStruct((Q * K,), jnp.float32),
              jax.ShapeDtypeStruct((Q * K,), jnp.int32)),
    mesh=_mesh,
    compiler_params=pltpu.CompilerParams(needs_layout_passes=False),
    scratch_types=[
        pltpu.VMEM((QW, 3), jnp.float32),
        pltpu.VMEM((3, J, PPAD), jnp.float32),
        pltpu.VMEM((32, 3), jnp.float32),
        pltpu.VMEM((QW * K + 8,), jnp.float32),
        pltpu.VMEM((QW * K + 8,), jnp.int32),
    ])
def _knn_sc(q_hbm, p_hbm, s_hbm, kd_hbm, ki_hbm,
            q_v, p_v, s_v, kd_v, ki_v):
  wid = lax.axis_index("s") * NC + lax.axis_index("c")
  qbase = wid * QW
  pltpu.sync_copy(q_hbm.at[pl.ds(qbase, QW)], q_v)
  pltpu.sync_copy(p_hbm, p_v)
  pltpu.sync_copy(s_hbm, s_v)

  iota = lax.iota(jnp.int32, L)
  cvec = [jnp.full((L,), c, jnp.int32) for c in range(3)]

  def scale_body(j, carry):
    jvec = jnp.full((L,), j, jnp.int32)
    for c in range(3):
      sc = plsc.load_gather(s_v, [jvec, cvec[c]])
      for k in range(NVREG):
        sl = pl.ds(k * 16, 16)
        p_v[c, j, sl] = p_v[c, j, sl] / sc
    return carry

  lax.fori_loop(0, J, scale_body, 0)

  zero = jnp.zeros((L,), jnp.float32)

  def ctr_body(j, carry):
    carry = list(carry)
    for c in range(3):
      acc = p_v[c, j, pl.ds(0, 16)]
      for k in range(1, NVREG - 1):
        acc = acc + p_v[c, j, pl.ds(k * 16, 16)]
      last = jnp.where(iota < 8, p_v[c, j, pl.ds((NVREG - 1) * 16, 16)], 0.0)
      acc = acc + last
      m = jnp.sum(acc) * jnp.float32(1.0 / P)
      for h in range(2):
        carry[c * 2 + h] = jnp.where(iota + h * 16 == j, m, carry[c * 2 + h])
    return tuple(carry)

  ctr = lax.fori_loop(0, J, ctr_body, (zero,) * 6)

  def q_body(i, carry):
    ivec = jnp.full((L,), i, jnp.int32)
    q = [plsc.load_gather(q_v, [ivec, cvec[c]]) for c in range(3)]

    halves = []
    for h in range(2):
      jlane = iota + h * 16
      d = None
      for c in range(3):
        sc = plsc.load_gather(s_v, [jlane, cvec[c]])
        t = q[c] / sc - ctr[c * 2 + h]
        d = t * t if d is None else d + t * t
      if h == 1:
        d = jnp.where(jlane < J, d, BIGF)
      halves.append((d, jlane))
    _, v3 = _top16(halves)

    runs = []
    for r in range(KNN_VOLS):
      jr_s = jnp.sum(jnp.where(iota == r, v3, 0))
      jrv = jnp.full((L,), jr_s, jnp.int32)
      qs = [q[c] / plsc.load_gather(s_v, [jrv, cvec[c]]) for c in range(3)]
      gbase = jrv * P
      for k in range(NVREG):
        sp = iota + k * 16
        d = None
        for c in range(3):
          g = p_v[c, jr_s, pl.ds(k * 16, 16)]
          t = qs[c] - g
          d = t * t if d is None else d + t * t
        runs.append((d, gbase + sp))
    bk, bv = _top16(runs)

    msk = iota < K
    plsc.store_compressed(kd_v.at[pl.ds(i * K, 16)], bk, mask=msk)
    plsc.store_compressed(ki_v.at[pl.ds(i * K, 16)], bv, mask=msk)
    return carry

  def q_pair(i2, carry):
    q_body(i2 * 2, carry)
    q_body(i2 * 2 + 1, carry)
    return carry

  lax.fori_loop(0, QW // 2, q_pair, 0)

  pltpu.sync_copy(kd_v.at[pl.ds(0, QW * K)], kd_hbm.at[pl.ds(qbase * K, QW * K)])
  pltpu.sync_copy(ki_v.at[pl.ds(0, QW * K)], ki_hbm.at[pl.ds(qbase * K, QW * K)])


def kernel(q_w, p_w, vol_scale):
  p_pad = jnp.full((3, J, PPAD), PAD_VAL, jnp.float32)
  p_pad = p_pad.at[:, :, :P].set(jnp.transpose(p_w, (2, 0, 1)))
  s_pad = jnp.ones((32, 3), jnp.float32).at[:J, :].set(vol_scale)
  kd, ki = _knn_sc(q_w, p_pad, s_pad)
  return kd.reshape(Q, K), ki.reshape(Q, K)


# speedup vs baseline: 1.6123x; 1.0487x over previous
import functools

import jax
import jax.numpy as jnp
from jax import lax
from jax.experimental import pallas as pl
from jax.experimental.pallas import tpu as pltpu
from jax.experimental.pallas import tpu_sc as plsc

J = 19
P = 200
PPAD = 208
NVREG = PPAD // 16
KNN_VOLS = 3
K = 8
Q = 16384
NC, NS, L = 2, 16, 16
NW = NC * NS
QW = Q // NW
PAD_VAL = 1.0e15
BIGF = 3.0e38


def _merge16(a, b):
  ak, av = a
  bk, bv = b
  rk = lax.rev(bk, (0,))
  rv = lax.rev(bv, (0,))
  take_a = ak <= rk
  mk = jnp.where(take_a, ak, rk)
  mv = jnp.where(take_a, av, rv)
  sk, sv = plsc.sort_key_val(mk, mv)
  return sk, sv


def _merge_tree(runs):
  while len(runs) > 1:
    nxt = [_merge16(runs[t], runs[t + 1]) for t in range(0, len(runs) - 1, 2)]
    if len(runs) % 2:
      nxt.append(runs[-1])
    runs = nxt
  return runs[0]


_mesh = plsc.VectorSubcoreMesh(
    core_axis_name="c", subcore_axis_name="s", num_cores=NC, num_subcores=NS)


@functools.partial(
    pl.kernel,
    out_type=(jax.ShapeDtypeStruct((Q * K,), jnp.float32),
              jax.ShapeDtypeStruct((Q * K,), jnp.int32)),
    mesh=_mesh,
    compiler_params=pltpu.CompilerParams(needs_layout_passes=False),
    scratch_types=[
        pltpu.VMEM((QW, 3), jnp.float32),
        pltpu.VMEM((3, J, PPAD), jnp.float32),
        pltpu.VMEM((32, 3), jnp.float32),
        pltpu.VMEM((QW * K + 8,), jnp.float32),
        pltpu.VMEM((QW * K + 8,), jnp.int32),
    ])
def _knn_sc(q_hbm, p_hbm, s_hbm, kd_hbm, ki_hbm,
            q_v, p_v, s_v, kd_v, ki_v):
  wid = lax.axis_index("s") * NC + lax.axis_index("c")
  qbase = wid * QW
  pltpu.sync_copy(q_hbm.at[pl.ds(qbase, QW)], q_v)
  pltpu.sync_copy(p_hbm, p_v)
  pltpu.sync_copy(s_hbm, s_v)

  iota = lax.iota(jnp.int32, L)
  cvec = [jnp.full((L,), c, jnp.int32) for c in range(3)]

  def scale_body(j, carry):
    jvec = jnp.full((L,), j, jnp.int32)
    for c in range(3):
      sc = plsc.load_gather(s_v, [jvec, cvec[c]])
      for k in range(NVREG):
        sl = pl.ds(k * 16, 16)
        p_v[c, j, sl] = p_v[c, j, sl] / sc
    return carry

  lax.fori_loop(0, J, scale_body, 0)

  zero = jnp.zeros((L,), jnp.float32)

  def ctr_body(j, carry):
    carry = list(carry)
    for c in range(3):
      acc = p_v[c, j, pl.ds(0, 16)]
      for k in range(1, NVREG - 1):
        acc = acc + p_v[c, j, pl.ds(k * 16, 16)]
      last = jnp.where(iota < 8, p_v[c, j, pl.ds((NVREG - 1) * 16, 16)], 0.0)
      acc = acc + last
      m = jnp.sum(acc) * jnp.float32(1.0 / P)
      for h in range(2):
        carry[c * 2 + h] = jnp.where(iota + h * 16 == j, m, carry[c * 2 + h])
    return tuple(carry)

  ctr = lax.fori_loop(0, J, ctr_body, (zero,) * 6)

  def q_body(i, carry):
    ivec = jnp.full((L,), i, jnp.int32)
    q = [plsc.load_gather(q_v, [ivec, cvec[c]]) for c in range(3)]

    halves = []
    for h in range(2):
      jlane = iota + h * 16
      d = None
      for c in range(3):
        sc = plsc.load_gather(s_v, [jlane, cvec[c]])
        t = q[c] / sc - ctr[c * 2 + h]
        d = t * t if d is None else d + t * t
      if h == 1:
        d = jnp.where(jlane < J, d, BIGF)
      halves.append(plsc.sort_key_val(d, jlane))
    _, v3 = _merge_tree(halves)

    runs = []
    for r in range(KNN_VOLS):
      jr_s = jnp.sum(jnp.where(iota == r, v3, 0))
      jrv = jnp.full((L,), jr_s, jnp.int32)
      qs = [q[c] / plsc.load_gather(s_v, [jrv, cvec[c]]) for c in range(3)]
      gbase = jrv * P
      for k in range(NVREG):
        sp = iota + k * 16
        d = None
        for c in range(3):
          g = p_v[c, jr_s, pl.ds(k * 16, 16)]
          t = qs[c] - g
          d = t * t if d is None else d + t * t
        runs.append(plsc.sort_key_val(d, gbase + sp))
    bk, bv = _merge_tree(runs)

    msk = iota < K
    plsc.store_compressed(kd_v.at[pl.ds(i * K, 16)], bk, mask=msk)
    plsc.store_compressed(ki_v.at[pl.ds(i * K, 16)], bv, mask=msk)
    return carry

  def q_pair(i2, carry):
    q_body(i2 * 2, carry)
    q_body(i2 * 2 + 1, carry)
    return carry

  lax.fori_loop(0, QW // 2, q_pair, 0)

  pltpu.sync_copy(kd_v.at[pl.ds(0, QW * K)], kd_hbm.at[pl.ds(qbase * K, QW * K)])
  pltpu.sync_copy(ki_v.at[pl.ds(0, QW * K)], ki_hbm.at[pl.ds(qbase * K, QW * K)])


def kernel(q_w, p_w, vol_scale):
  p_pad = jnp.full((3, J, PPAD), PAD_VAL, jnp.float32)
  p_pad = p_pad.at[:, :, :P].set(jnp.transpose(p_w, (2, 0, 1)))
  s_pad = jnp.ones((32, 3), jnp.float32).at[:J, :].set(vol_scale)
  kd, ki = _knn_sc(q_w, p_pad, s_pad)
  return kd.reshape(Q, K), ki.reshape(Q, K)


# speedup vs baseline: 1.7769x; 1.1021x over previous
import functools

import jax
import jax.numpy as jnp
from jax import lax
from jax.experimental import pallas as pl
from jax.experimental.pallas import tpu as pltpu
from jax.experimental.pallas import tpu_sc as plsc

J = 19
P = 200
PPAD = 208
NVREG = PPAD // 16
KNN_VOLS = 3
K = 8
Q = 16384
NC, NS, L = 2, 16, 16
NW = NC * NS
QW = Q // NW
PAD_VAL = 1.0e15
BIGF = 3.0e38


def _merge16(a, b):
  ak, av = a
  bk, bv = b
  rk = lax.rev(bk, (0,))
  rv = lax.rev(bv, (0,))
  take_a = ak <= rk
  mk = jnp.where(take_a, ak, rk)
  mv = jnp.where(take_a, av, rv)
  sk, sv = plsc.sort_key_val(mk, mv)
  return sk, sv


def _merge_tree(runs):
  while len(runs) > 1:
    nxt = [_merge16(runs[t], runs[t + 1]) for t in range(0, len(runs) - 1, 2)]
    if len(runs) % 2:
      nxt.append(runs[-1])
    runs = nxt
  return runs[0]


_mesh = plsc.VectorSubcoreMesh(
    core_axis_name="c", subcore_axis_name="s", num_cores=NC, num_subcores=NS)


@functools.partial(
    pl.kernel,
    out_type=(jax.ShapeDtypeStruct((Q * K,), jnp.float32),
              jax.ShapeDtypeStruct((Q * K,), jnp.int32)),
    mesh=_mesh,
    compiler_params=pltpu.CompilerParams(needs_layout_passes=False),
    scratch_types=[
        pltpu.VMEM((QW, 3), jnp.float32),
        pltpu.VMEM((3, J, PPAD), jnp.float32),
        pltpu.VMEM((32, 3), jnp.float32),
        pltpu.VMEM((QW * K + 8,), jnp.float32),
        pltpu.VMEM((QW * K + 8,), jnp.int32),
    ])
def _knn_sc(q_hbm, p_hbm, s_hbm, kd_hbm, ki_hbm,
            q_v, p_v, s_v, kd_v, ki_v):
  wid = lax.axis_index("s") * NC + lax.axis_index("c")
  qbase = wid * QW
  pltpu.sync_copy(q_hbm.at[pl.ds(qbase, QW)], q_v)
  pltpu.sync_copy(p_hbm, p_v)
  pltpu.sync_copy(s_hbm, s_v)

  iota = lax.iota(jnp.int32, L)
  cvec = [jnp.full((L,), c, jnp.int32) for c in range(3)]

  def scale_body(j, carry):
    jvec = jnp.full((L,), j, jnp.int32)
    for c in range(3):
      sc = plsc.load_gather(s_v, [jvec, cvec[c]])
      for k in range(NVREG):
        sl = pl.ds(k * 16, 16)
        p_v[c, j, sl] = p_v[c, j, sl] / sc
    return carry

  lax.fori_loop(0, J, scale_body, 0)

  zero = jnp.full((L,), 5.0e18, jnp.float32)

  def ctr_body(j, carry):
    carry = list(carry)
    for c in range(3):
      acc = p_v[c, j, pl.ds(0, 16)]
      for k in range(1, NVREG - 1):
        acc = acc + p_v[c, j, pl.ds(k * 16, 16)]
      last = jnp.where(iota < 8, p_v[c, j, pl.ds((NVREG - 1) * 16, 16)], 0.0)
      acc = acc + last
      m = jnp.sum(acc) * jnp.float32(1.0 / P)
      for h in range(2):
        carry[c * 2 + h] = jnp.where(iota + h * 16 == j, m, carry[c * 2 + h])
    return tuple(carry)

  ctr = lax.fori_loop(0, J, ctr_body, (zero,) * 6)

  inv = [jnp.float32(1.0) / plsc.load_gather(s_v, [iota + (c // 3) * 16,
                                                   cvec[c % 3]])
         for c in range(6)]

  def q_body(i, carry):
    ivec = jnp.full((L,), i, jnp.int32)
    q = [plsc.load_gather(q_v, [ivec, cvec[c]]) for c in range(3)]

    halves = []
    for h in range(2):
      d = None
      for c in range(3):
        t = q[c] * inv[h * 3 + c] - ctr[c * 2 + h]
        d = t * t if d is None else d + t * t
      halves.append(plsc.sort_key_val(d, iota + h * 16))
    _, v3 = _merge_tree(halves)

    runs = []
    for r in range(KNN_VOLS):
      jr_s = jnp.sum(jnp.where(iota == r, v3, 0))
      jrv = jnp.full((L,), jr_s, jnp.int32)
      qs = [q[c] / plsc.load_gather(s_v, [jrv, cvec[c]]) for c in range(3)]
      gbase = jrv * P
      for k in range(NVREG):
        sp = iota + k * 16
        d = None
        for c in range(3):
          g = p_v[c, jr_s, pl.ds(k * 16, 16)]
          t = qs[c] - g
          d = t * t if d is None else d + t * t
        runs.append(plsc.sort_key_val(d, gbase + sp))
    bk, bv = _merge_tree(runs)

    msk = iota < K
    plsc.store_compressed(kd_v.at[pl.ds(i * K, 16)], bk, mask=msk)
    plsc.store_compressed(ki_v.at[pl.ds(i * K, 16)], bv, mask=msk)
    return carry

  def q_pair(i2, carry):
    q_body(i2 * 2, carry)
    q_body(i2 * 2 + 1, carry)
    return carry

  lax.fori_loop(0, QW // 2, q_pair, 0)

  pltpu.sync_copy(kd_v.at[pl.ds(0, QW * K)], kd_hbm.at[pl.ds(qbase * K, QW * K)])
  pltpu.sync_copy(ki_v.at[pl.ds(0, QW * K)], ki_hbm.at[pl.ds(qbase * K, QW * K)])


def kernel(q_w, p_w, vol_scale):
  p_pad = jnp.full((3, J, PPAD), PAD_VAL, jnp.float32)
  p_pad = p_pad.at[:, :, :P].set(jnp.transpose(p_w, (2, 0, 1)))
  s_pad = jnp.ones((32, 3), jnp.float32).at[:J, :].set(vol_scale)
  kd, ki = _knn_sc(q_w, p_pad, s_pad)
  return kd.reshape(Q, K), ki.reshape(Q, K)


# speedup vs baseline: 1.8286x; 1.0291x over previous
import functools

import jax
import jax.numpy as jnp
from jax import lax
from jax.experimental import pallas as pl
from jax.experimental.pallas import tpu as pltpu
from jax.experimental.pallas import tpu_sc as plsc

J = 19
P = 200
PPAD = 208
NVREG = PPAD // 16
KNN_VOLS = 3
K = 8
Q = 16384
NC, NS, L = 2, 16, 16
NW = NC * NS
QW = Q // NW
PAD_VAL = 1.0e15
BIGF = 3.0e38


def _merge16(a, b):
  ak, av = a
  bk, bv = b
  rk = lax.rev(bk, (0,))
  rv = lax.rev(bv, (0,))
  take_a = ak <= rk
  mk = jnp.where(take_a, ak, rk)
  mv = jnp.where(take_a, av, rv)
  sk, sv = plsc.sort_key_val(mk, mv)
  return sk, sv


def _merge_tree(runs):
  while len(runs) > 1:
    nxt = [_merge16(runs[t], runs[t + 1]) for t in range(0, len(runs) - 1, 2)]
    if len(runs) % 2:
      nxt.append(runs[-1])
    runs = nxt
  return runs[0]


_mesh = plsc.VectorSubcoreMesh(
    core_axis_name="c", subcore_axis_name="s", num_cores=NC, num_subcores=NS)


@functools.partial(
    pl.kernel,
    out_type=(jax.ShapeDtypeStruct((Q * K,), jnp.float32),
              jax.ShapeDtypeStruct((Q * K,), jnp.int32)),
    mesh=_mesh,
    compiler_params=pltpu.CompilerParams(needs_layout_passes=False),
    scratch_types=[
        pltpu.VMEM((QW, 3), jnp.float32),
        pltpu.VMEM((3, J, PPAD), jnp.float32),
        pltpu.VMEM((32, 3), jnp.float32),
        pltpu.VMEM((QW * K + 8,), jnp.float32),
        pltpu.VMEM((QW * K + 8,), jnp.int32),
    ])
def _knn_sc(q_hbm, p_hbm, s_hbm, kd_hbm, ki_hbm,
            q_v, p_v, s_v, kd_v, ki_v):
  wid = lax.axis_index("s") * NC + lax.axis_index("c")
  qbase = wid * QW
  pltpu.sync_copy(q_hbm.at[pl.ds(qbase, QW)], q_v)
  pltpu.sync_copy(p_hbm, p_v)
  pltpu.sync_copy(s_hbm, s_v)

  iota = lax.iota(jnp.int32, L)
  cvec = [jnp.full((L,), c, jnp.int32) for c in range(3)]

  def scale_body(j, carry):
    jvec = jnp.full((L,), j, jnp.int32)
    for c in range(3):
      sc = plsc.load_gather(s_v, [jvec, cvec[c]])
      for k in range(NVREG):
        sl = pl.ds(k * 16, 16)
        p_v[c, j, sl] = p_v[c, j, sl] / sc
    return carry

  lax.fori_loop(0, J, scale_body, 0)

  zero = jnp.full((L,), 5.0e18, jnp.float32)

  def ctr_body(j, carry):
    carry = list(carry)
    for c in range(3):
      acc = p_v[c, j, pl.ds(0, 16)]
      for k in range(1, NVREG - 1):
        acc = acc + p_v[c, j, pl.ds(k * 16, 16)]
      last = jnp.where(iota < 8, p_v[c, j, pl.ds((NVREG - 1) * 16, 16)], 0.0)
      acc = acc + last
      m = jnp.sum(acc) * jnp.float32(1.0 / P)
      for h in range(2):
        carry[c * 2 + h] = jnp.where(iota + h * 16 == j, m, carry[c * 2 + h])
    return tuple(carry)

  ctr = lax.fori_loop(0, J, ctr_body, (zero,) * 6)

  inv = [jnp.float32(1.0) / plsc.load_gather(s_v, [iota + (c // 3) * 16,
                                                   cvec[c % 3]])
         for c in range(6)]

  def q_body(i, carry):
    ivec = jnp.full((L,), i, jnp.int32)
    q = [plsc.load_gather(q_v, [ivec, cvec[c]]) for c in range(3)]

    halves = []
    for h in range(2):
      d = None
      for c in range(3):
        t = q[c] * inv[h * 3 + c] - ctr[c * 2 + h]
        d = t * t if d is None else d + t * t
      halves.append(plsc.sort_key_val(d, iota + h * 16))
    _, v3 = _merge_tree(halves)

    runs = []
    for r in range(KNN_VOLS):
      jr_s = v3[r]
      jrv = jnp.full((L,), jr_s, jnp.int32)
      qs = [q[c] / plsc.load_gather(s_v, [jrv, cvec[c]]) for c in range(3)]
      gbase = jrv * P
      for k in range(NVREG):
        sp = iota + k * 16
        d = None
        for c in range(3):
          g = p_v[c, jr_s, pl.ds(k * 16, 16)]
          t = qs[c] - g
          d = t * t if d is None else d + t * t
        runs.append(plsc.sort_key_val(d, gbase + sp))
    bk, bv = _merge_tree(runs)

    msk = iota < K
    plsc.store_compressed(kd_v.at[pl.ds(i * K, 16)], bk, mask=msk)
    plsc.store_compressed(ki_v.at[pl.ds(i * K, 16)], bv, mask=msk)
    return carry

  def q_pair(i2, carry):
    q_body(i2 * 2, carry)
    q_body(i2 * 2 + 1, carry)
    return carry

  lax.fori_loop(0, QW // 2, q_pair, 0)

  pltpu.sync_copy(kd_v.at[pl.ds(0, QW * K)], kd_hbm.at[pl.ds(qbase * K, QW * K)])
  pltpu.sync_copy(ki_v.at[pl.ds(0, QW * K)], ki_hbm.at[pl.ds(qbase * K, QW * K)])


def kernel(q_w, p_w, vol_scale):
  p_pad = jnp.full((3, J, PPAD), PAD_VAL, jnp.float32)
  p_pad = p_pad.at[:, :, :P].set(jnp.transpose(p_w, (2, 0, 1)))
  s_pad = jnp.ones((32, 3), jnp.float32).at[:J, :].set(vol_scale)
  kd, ki = _knn_sc(q_w, p_pad, s_pad)
  return kd.reshape(Q, K), ki.reshape(Q, K)


# speedup vs baseline: 1.9607x; 1.0723x over previous
import functools

import jax
import jax.numpy as jnp
from jax import lax
from jax.experimental import pallas as pl
from jax.experimental.pallas import tpu as pltpu
from jax.experimental.pallas import tpu_sc as plsc

J = 19
P = 200
PPAD = 208
NVREG = PPAD // 16
KNN_VOLS = 3
K = 8
Q = 16384
NC, NS, L = 2, 16, 16
NW = NC * NS
QW = Q // NW
PAD_VAL = 1.0e15
BIGF = 3.0e38


def _merge16(a, b):
  ak, av = a
  bk, bv = b
  rk = lax.rev(bk, (0,))
  rv = lax.rev(bv, (0,))
  take_a = ak <= rk
  mk = jnp.where(take_a, ak, rk)
  mv = jnp.where(take_a, av, rv)
  sk, sv = plsc.sort_key_val(mk, mv)
  return sk, sv


def _merge_tree(runs):
  while len(runs) > 1:
    nxt = [_merge16(runs[t], runs[t + 1]) for t in range(0, len(runs) - 1, 2)]
    if len(runs) % 2:
      nxt.append(runs[-1])
    runs = nxt
  return runs[0]


_mesh = plsc.VectorSubcoreMesh(
    core_axis_name="c", subcore_axis_name="s", num_cores=NC, num_subcores=NS)


@functools.partial(
    pl.kernel,
    out_type=(jax.ShapeDtypeStruct((Q * K,), jnp.float32),
              jax.ShapeDtypeStruct((Q * K,), jnp.int32)),
    mesh=_mesh,
    compiler_params=pltpu.CompilerParams(needs_layout_passes=False),
    scratch_types=[
        pltpu.VMEM((QW, 3), jnp.float32),
        pltpu.VMEM((3, J, PPAD), jnp.float32),
        pltpu.VMEM((32, 3), jnp.float32),
        pltpu.VMEM((QW * K + 8,), jnp.float32),
        pltpu.VMEM((QW * K + 8,), jnp.int32),
    ])
def _knn_sc(q_hbm, p_hbm, s_hbm, kd_hbm, ki_hbm,
            q_v, p_v, s_v, kd_v, ki_v):
  wid = lax.axis_index("s") * NC + lax.axis_index("c")
  qbase = wid * QW
  pltpu.sync_copy(q_hbm.at[pl.ds(qbase, QW)], q_v)
  pltpu.sync_copy(p_hbm, p_v)
  pltpu.sync_copy(s_hbm, s_v)

  iota = lax.iota(jnp.int32, L)
  cvec = [jnp.full((L,), c, jnp.int32) for c in range(3)]

  def scale_body(j, carry):
    jvec = jnp.full((L,), j, jnp.int32)
    for c in range(3):
      sc = plsc.load_gather(s_v, [jvec, cvec[c]])
      for k in range(NVREG):
        sl = pl.ds(k * 16, 16)
        p_v[c, j, sl] = p_v[c, j, sl] / sc
    return carry

  lax.fori_loop(0, J, scale_body, 0)

  zero = jnp.full((L,), 5.0e18, jnp.float32)

  def ctr_body(j, carry):
    carry = list(carry)
    for c in range(3):
      acc = p_v[c, j, pl.ds(0, 16)]
      for k in range(1, NVREG - 1):
        acc = acc + p_v[c, j, pl.ds(k * 16, 16)]
      last = jnp.where(iota < 8, p_v[c, j, pl.ds((NVREG - 1) * 16, 16)], 0.0)
      acc = acc + last
      m = jnp.sum(acc) * jnp.float32(1.0 / P)
      for h in range(2):
        carry[c * 2 + h] = jnp.where(iota + h * 16 == j, m, carry[c * 2 + h])
    return tuple(carry)

  ctr = lax.fori_loop(0, J, ctr_body, (zero,) * 6)

  inv = [jnp.float32(1.0) / plsc.load_gather(s_v, [iota + (c // 3) * 16,
                                                   cvec[c % 3]])
         for c in range(6)]

  def stage1(i):
    ivec = jnp.full((L,), i, jnp.int32)
    q = [plsc.load_gather(q_v, [ivec, cvec[c]]) for c in range(3)]
    halves = []
    for h in range(2):
      d = None
      for c in range(3):
        t = q[c] * inv[h * 3 + c] - ctr[c * 2 + h]
        d = t * t if d is None else d + t * t
      halves.append(plsc.sort_key_val(d, iota + h * 16))
    _, v3 = _merge_tree(halves)
    return v3, q[0], q[1], q[2]

  def q_body(i, carry):
    v3, q0, q1, q2 = carry
    q = (q0, q1, q2)
    nxt = stage1(jnp.minimum(i + 1, QW - 1))

    runs = []
    for r in range(KNN_VOLS):
      jr_s = v3[r]
      jrv = jnp.full((L,), jr_s, jnp.int32)
      qs = [q[c] / plsc.load_gather(s_v, [jrv, cvec[c]]) for c in range(3)]
      gbase = jrv * P
      for k in range(NVREG):
        sp = iota + k * 16
        d = None
        for c in range(3):
          g = p_v[c, jr_s, pl.ds(k * 16, 16)]
          t = qs[c] - g
          d = t * t if d is None else d + t * t
        runs.append(plsc.sort_key_val(d, gbase + sp))
    bk, bv = _merge_tree(runs)

    msk = iota < K
    plsc.store_compressed(kd_v.at[pl.ds(i * K, 16)], bk, mask=msk)
    plsc.store_compressed(ki_v.at[pl.ds(i * K, 16)], bv, mask=msk)
    return nxt

  lax.fori_loop(0, QW, q_body, stage1(0))

  pltpu.sync_copy(kd_v.at[pl.ds(0, QW * K)], kd_hbm.at[pl.ds(qbase * K, QW * K)])
  pltpu.sync_copy(ki_v.at[pl.ds(0, QW * K)], ki_hbm.at[pl.ds(qbase * K, QW * K)])


def kernel(q_w, p_w, vol_scale):
  p_pad = jnp.full((3, J, PPAD), PAD_VAL, jnp.float32)
  p_pad = p_pad.at[:, :, :P].set(jnp.transpose(p_w, (2, 0, 1)))
  s_pad = jnp.ones((32, 3), jnp.float32).at[:J, :].set(vol_scale)
  kd, ki = _knn_sc(q_w, p_pad, s_pad)
  return kd.reshape(Q, K), ki.reshape(Q, K)
